# baseline (speedup 1.0000x reference)
import dataclasses
import functools

import jax
import jax.numpy as jnp
from jax import lax
from jax.experimental import pallas as pl
from jax.experimental.pallas import tpu as pltpu
from jax.experimental.pallas import tpu_sc as plsc

N_NODES = 10000
N_FEAT = 128
N_EDGES = 320000

NC = 2
NS = 16
NW = NC * NS

NP = 10112
CH = 128
CPT = -(-N_EDGES // (NW * CH))
CPT += CPT % 2
EP = NW * CH * CPT
G = 16
ROWS_PT = NP // NS

_mesh = plsc.VectorSubcoreMesh(core_axis_name="c", subcore_axis_name="s")

_cp = pltpu.CompilerParams()
if "needs_layout_passes" in pltpu.CompilerParams.__dataclass_fields__:
    _cp = dataclasses.replace(_cp, needs_layout_passes=False)


@functools.partial(
    pl.kernel,
    mesh=_mesh,
    out_type=jax.ShapeDtypeStruct((NW * NP,), jnp.float32),
    compiler_params=_cp,
    scratch_types=[
        pltpu.VMEM((CPT, CH), jnp.int32),
        pltpu.VMEM((NP,), jnp.float32),
    ],
)
def _sc_degree(dst_hbm, out_hbm, dst_v, hist):
    c = lax.axis_index("c")
    s = lax.axis_index("s")
    w = s * NC + c

    pltpu.sync_copy(dst_hbm.at[w], dst_v)

    zero = jnp.zeros((16,), jnp.float32)
    one = jnp.ones((16,), jnp.float32)

    @pl.loop(0, NP // 16)
    def _(i):
        hist[pl.ds(i * 16, 16)] = zero

    @pl.loop(0, CPT)
    def _(j):
        @pl.loop(0, CH // 16)
        def _(k):
            idx = dst_v[j, pl.ds(k * 16, 16)]
            plsc.addupdate_scatter(hist, [idx], one)

    pltpu.sync_copy(hist, out_hbm.at[pl.ds(w * NP, NP)])


@functools.partial(
    pl.kernel,
    mesh=_mesh,
    out_type=jax.ShapeDtypeStruct((NC, NP, N_FEAT), jnp.float32),
    scratch_types=[
        pltpu.VMEM((G, CH), jnp.int32),
        pltpu.VMEM((G, CH), jnp.int32),
        pltpu.VMEM((G, CH), jnp.int32),
        pltpu.VMEM((G, CH), jnp.int32),
        pltpu.VMEM((CH, N_FEAT), jnp.float32),
        pltpu.VMEM((CH, N_FEAT), jnp.float32),
        pltpu.VMEM_SHARED((NP, N_FEAT), jnp.float32),
        pltpu.SemaphoreType.DMA,
        pltpu.SemaphoreType.DMA,
        pltpu.SemaphoreType.DMA,
    ],
)
def _sc_scatter(h_hbm, src_hbm, dst_hbm, out_hbm, s0_v, d0_v, s1_v, d1_v,
                bufa, bufb, acc, sema, semb, isem):
    c = lax.axis_index("c")
    s = lax.axis_index("s")
    w = s * NC + c
    r0 = s * ROWS_PT

    NG = CPT // G
    idx_bufs = ((s0_v, d0_v), (s1_v, d1_v))

    pltpu.async_copy(src_hbm.at[w, pl.ds(0, G)], s0_v, isem)
    pltpu.async_copy(dst_hbm.at[w, pl.ds(0, G)], d0_v, isem)

    @pl.when(c == 0)
    def _():
        pltpu.sync_copy(h_hbm.at[pl.ds(r0, ROWS_PT)],
                        acc.at[pl.ds(r0, ROWS_PT)])

    @pl.when(c != 0)
    def _():
        zero = jnp.zeros((16,), jnp.float32)

        @pl.loop(0, CH)
        def _(i):
            @pl.loop(0, N_FEAT // 16)
            def _(j):
                bufa[i, pl.ds(j * 16, 16)] = zero

        n_full = ROWS_PT // CH
        rem = ROWS_PT - n_full * CH
        for t in range(n_full):
            pltpu.sync_copy(bufa, acc.at[pl.ds(r0 + t * CH, CH)])
        if rem:
            pltpu.sync_copy(bufa.at[pl.ds(0, rem)],
                            acc.at[pl.ds(r0 + n_full * CH, rem)])

    pltpu.make_async_copy(src_hbm.at[w, pl.ds(0, G)], s0_v, isem).wait()
    pltpu.make_async_copy(dst_hbm.at[w, pl.ds(0, G)], d0_v, isem).wait()
    pltpu.async_copy(h_hbm.at[s0_v.at[0]], bufb, semb)

    plsc.subcore_barrier()

    for g in range(NG):
        sg, dg = idx_bufs[g % 2]
        sn, dn = idx_bufs[(g + 1) % 2]
        if g + 1 < NG:
            pltpu.async_copy(src_hbm.at[w, pl.ds((g + 1) * G, G)], sn, isem)
            pltpu.async_copy(dst_hbm.at[w, pl.ds((g + 1) * G, G)], dn, isem)

        @pl.loop(0, G - 2, step=2)
        def _(j):
            pltpu.async_copy(h_hbm.at[sg.at[j + 1]], bufa, sema)
            pltpu.make_async_copy(h_hbm.at[sg.at[j]], bufb, semb).wait()
            pltpu.sync_copy(bufb, acc.at[dg.at[j]], add=True)
            pltpu.async_copy(h_hbm.at[sg.at[j + 2]], bufb, semb)
            pltpu.make_async_copy(h_hbm.at[sg.at[j + 1]], bufa, sema).wait()
            pltpu.sync_copy(bufa, acc.at[dg.at[j + 1]], add=True)

        pltpu.async_copy(h_hbm.at[sg.at[G - 1]], bufa, sema)
        pltpu.make_async_copy(h_hbm.at[sg.at[G - 2]], bufb, semb).wait()
        pltpu.sync_copy(bufb, acc.at[dg.at[G - 2]], add=True)
        if g + 1 < NG:
            pltpu.make_async_copy(
                src_hbm.at[w, pl.ds((g + 1) * G, G)], sn, isem).wait()
            pltpu.make_async_copy(
                dst_hbm.at[w, pl.ds((g + 1) * G, G)], dn, isem).wait()
            pltpu.async_copy(h_hbm.at[sn.at[0]], bufb, semb)
        pltpu.make_async_copy(h_hbm.at[sg.at[G - 1]], bufa, sema).wait()
        pltpu.sync_copy(bufa, acc.at[dg.at[G - 1]], add=True)

    plsc.subcore_barrier()

    pltpu.sync_copy(acc.at[pl.ds(r0, ROWS_PT)],
                    out_hbm.at[c, pl.ds(r0, ROWS_PT)])


NB = 8
RB = NP // NB

_row = lambda bs: pl.BlockSpec(bs, lambda i: (0,) * (len(bs) - 2) + (i, 0))
_rep = lambda bs: pl.BlockSpec(bs, lambda i: (0,) * len(bs))


def _tc_l1(degm, x_p, W1):
    def body(p_ref, x_ref, w_ref, dinv_ref, hp_ref):
        deg = 1.0 + jnp.sum(p_ref[...], axis=1, keepdims=True)
        dinv = jnp.broadcast_to(lax.rsqrt(deg), (RB, N_FEAT))
        dinv_ref[...] = dinv
        hp_ref[...] = dinv * jnp.dot(x_ref[...], w_ref[...],
                                     preferred_element_type=jnp.float32)

    return pl.pallas_call(
        body,
        grid=(NB,),
        in_specs=[_row((RB, NW)), _row((RB, N_FEAT)),
                  _rep((N_FEAT, N_FEAT))],
        out_specs=(_row((RB, N_FEAT)), _row((RB, N_FEAT))),
        out_shape=(
            jax.ShapeDtypeStruct((NP, N_FEAT), jnp.float32),
            jax.ShapeDtypeStruct((NP, N_FEAT), jnp.float32),
        ),
    )(degm, x_p, W1)


def _tc_mid(parts, dinv, b1, W2):
    def body(p_ref, d_ref, b_ref, w_ref, o_ref):
        db = d_ref[...]
        z = db * (p_ref[0] + p_ref[1]) + b_ref[...]
        sp = jnp.maximum(z, 0.0) + jnp.log1p(jnp.exp(-jnp.abs(z)))
        o_ref[...] = db * jnp.dot(sp, w_ref[...],
                                  preferred_element_type=jnp.float32)

    return pl.pallas_call(
        body,
        grid=(NB,),
        in_specs=[_row((NC, RB, N_FEAT)), _row((RB, N_FEAT)),
                  _rep((1, N_FEAT)), _rep((N_FEAT, N_FEAT))],
        out_specs=_row((RB, N_FEAT)),
        out_shape=jax.ShapeDtypeStruct((NP, N_FEAT), jnp.float32),
    )(parts, dinv, b1, W2)


def _tc_final(parts, dinv, b2):
    def body(p_ref, d_ref, b_ref, o_ref):
        o_ref[...] = d_ref[...] * (p_ref[0] + p_ref[1]) + b_ref[...]

    return pl.pallas_call(
        body,
        grid=(NB,),
        in_specs=[_row((NC, RB, N_FEAT)), _row((RB, N_FEAT)),
                  _rep((1, N_FEAT))],
        out_specs=_row((RB, N_FEAT)),
        out_shape=jax.ShapeDtypeStruct((NP, N_FEAT), jnp.float32),
    )(parts, dinv, b2)


def kernel(x, edge_index, W1, b1, W2, b2):
    src = edge_index[0]
    dst = edge_index[1]
    pad = EP - N_EDGES
    pad_src = jnp.arange(pad, dtype=jnp.int32) % N_NODES
    pad_dst = N_NODES + jnp.arange(pad, dtype=jnp.int32) % (NP - N_NODES)
    src_t = jnp.concatenate([src, pad_src]).reshape(NW, CPT, CH)
    dst_t = jnp.concatenate([dst, pad_dst]).reshape(NW, CPT, CH)

    x_p = jnp.pad(x, ((0, NP - N_NODES), (0, 0)))
    b1r = b1.reshape(1, N_FEAT)
    b2r = b2.reshape(1, N_FEAT)

    deg_parts = _sc_degree(dst_t)
    degm = deg_parts.reshape(NW, NP).T
    dinv, h1p = _tc_l1(degm, x_p, W1)
    parts1 = _sc_scatter(h1p, src_t, dst_t)
    h2p = _tc_mid(parts1, dinv, b1r, W2)
    parts2 = _sc_scatter(h2p, src_t, dst_t)
    out = _tc_final(parts2, dinv, b2r)
    return out[:N_NODES]


# speedup vs baseline: 1.0158x; 1.0158x over previous
import dataclasses
import functools

import jax
import jax.numpy as jnp
from jax import lax
from jax.experimental import pallas as pl
from jax.experimental.pallas import tpu as pltpu
from jax.experimental.pallas import tpu_sc as plsc

N_NODES = 10000
N_FEAT = 128
N_EDGES = 320000

NC = 2
NS = 16
NW = NC * NS

NP = 10112
CH = 128
CPT = -(-N_EDGES // (NW * CH))
CPT += CPT % 2
EP = NW * CH * CPT
G = 16
ROWS_PT = NP // NS

_mesh = plsc.VectorSubcoreMesh(core_axis_name="c", subcore_axis_name="s")

_cp = pltpu.CompilerParams()
if "needs_layout_passes" in pltpu.CompilerParams.__dataclass_fields__:
    _cp = dataclasses.replace(_cp, needs_layout_passes=False)


@functools.partial(
    pl.kernel,
    mesh=_mesh,
    out_type=jax.ShapeDtypeStruct((NW * NP,), jnp.float32),
    compiler_params=_cp,
    scratch_types=[
        pltpu.VMEM((CPT, CH), jnp.int32),
        pltpu.VMEM((NP,), jnp.float32),
    ],
)
def _sc_degree(dst_hbm, out_hbm, dst_v, hist):
    c = lax.axis_index("c")
    s = lax.axis_index("s")
    w = s * NC + c

    pltpu.sync_copy(dst_hbm.at[w], dst_v)

    zero = jnp.zeros((16,), jnp.float32)
    one = jnp.ones((16,), jnp.float32)

    @pl.loop(0, NP // 16)
    def _(i):
        hist[pl.ds(i * 16, 16)] = zero

    @pl.loop(0, CPT)
    def _(j):
        @pl.loop(0, CH // 16)
        def _(k):
            idx = dst_v[j, pl.ds(k * 16, 16)]
            plsc.addupdate_scatter(hist, [idx], one)

    pltpu.sync_copy(hist, out_hbm.at[pl.ds(w * NP, NP)])


@functools.partial(
    pl.kernel,
    mesh=_mesh,
    out_type=jax.ShapeDtypeStruct((NC, NP, N_FEAT), jnp.float32),
    scratch_types=[
        pltpu.VMEM((G, CH), jnp.int32),
        pltpu.VMEM((G, CH), jnp.int32),
        pltpu.VMEM((G, CH), jnp.int32),
        pltpu.VMEM((G, CH), jnp.int32),
        pltpu.VMEM((CH, N_FEAT), jnp.float32),
        pltpu.VMEM((CH, N_FEAT), jnp.float32),
        pltpu.VMEM_SHARED((NP, N_FEAT), jnp.float32),
        pltpu.SemaphoreType.DMA,
        pltpu.SemaphoreType.DMA,
        pltpu.SemaphoreType.DMA,
    ],
)
def _sc_scatter(h_hbm, src_hbm, dst_hbm, out_hbm, s0_v, d0_v, s1_v, d1_v,
                bufa, bufb, acc, sema, semb, isem):
    c = lax.axis_index("c")
    s = lax.axis_index("s")
    w = s * NC + c
    r0 = s * ROWS_PT

    NG = CPT // G
    idx_bufs = ((s0_v, d0_v), (s1_v, d1_v))

    pltpu.async_copy(src_hbm.at[w, pl.ds(0, G)], s0_v, isem)
    pltpu.async_copy(dst_hbm.at[w, pl.ds(0, G)], d0_v, isem)

    @pl.when(c == 0)
    def _():
        pltpu.sync_copy(h_hbm.at[pl.ds(r0, ROWS_PT)],
                        acc.at[pl.ds(r0, ROWS_PT)])

    @pl.when(c != 0)
    def _():
        zero = jnp.zeros((16,), jnp.float32)

        @pl.loop(0, CH)
        def _(i):
            @pl.loop(0, N_FEAT // 16)
            def _(j):
                bufa[i, pl.ds(j * 16, 16)] = zero

        n_full = ROWS_PT // CH
        rem = ROWS_PT - n_full * CH
        for t in range(n_full):
            pltpu.sync_copy(bufa, acc.at[pl.ds(r0 + t * CH, CH)])
        if rem:
            pltpu.sync_copy(bufa.at[pl.ds(0, rem)],
                            acc.at[pl.ds(r0 + n_full * CH, rem)])

    pltpu.make_async_copy(src_hbm.at[w, pl.ds(0, G)], s0_v, isem).wait()
    pltpu.make_async_copy(dst_hbm.at[w, pl.ds(0, G)], d0_v, isem).wait()
    pltpu.async_copy(h_hbm.at[s0_v.at[0]], bufb, semb)

    plsc.subcore_barrier()

    for g in range(NG):
        sg, dg = idx_bufs[g % 2]
        sn, dn = idx_bufs[(g + 1) % 2]
        if g + 1 < NG:
            pltpu.async_copy(src_hbm.at[w, pl.ds((g + 1) * G, G)], sn, isem)
            pltpu.async_copy(dst_hbm.at[w, pl.ds((g + 1) * G, G)], dn, isem)

        @pl.loop(0, G - 2, step=2)
        def _(j):
            pltpu.async_copy(h_hbm.at[sg.at[j + 1]], bufa, sema)
            pltpu.make_async_copy(h_hbm.at[sg.at[j]], bufb, semb).wait()
            pltpu.sync_copy(bufb, acc.at[dg.at[j]], add=True)
            pltpu.async_copy(h_hbm.at[sg.at[j + 2]], bufb, semb)
            pltpu.make_async_copy(h_hbm.at[sg.at[j + 1]], bufa, sema).wait()
            pltpu.sync_copy(bufa, acc.at[dg.at[j + 1]], add=True)

        pltpu.async_copy(h_hbm.at[sg.at[G - 1]], bufa, sema)
        pltpu.make_async_copy(h_hbm.at[sg.at[G - 2]], bufb, semb).wait()
        pltpu.sync_copy(bufb, acc.at[dg.at[G - 2]], add=True)
        if g + 1 < NG:
            pltpu.make_async_copy(
                src_hbm.at[w, pl.ds((g + 1) * G, G)], sn, isem).wait()
            pltpu.make_async_copy(
                dst_hbm.at[w, pl.ds((g + 1) * G, G)], dn, isem).wait()
            pltpu.async_copy(h_hbm.at[sn.at[0]], bufb, semb)
        pltpu.make_async_copy(h_hbm.at[sg.at[G - 1]], bufa, sema).wait()
        pltpu.sync_copy(bufa, acc.at[dg.at[G - 1]], add=True)

    plsc.subcore_barrier()

    pltpu.sync_copy(acc.at[pl.ds(r0, ROWS_PT)],
                    out_hbm.at[c, pl.ds(r0, ROWS_PT)])


NB = 8
RB = NP // NB

_row = lambda bs: pl.BlockSpec(bs, lambda i: (0,) * (len(bs) - 2) + (i, 0))
_rep = lambda bs: pl.BlockSpec(bs, lambda i: (0,) * len(bs))


def _tc_l1(degm, x_p, W1):
    def body(p_ref, x_ref, w_ref, dinv_ref, hp_ref):
        deg = 1.0 + jnp.sum(p_ref[...], axis=1, keepdims=True)
        dinv = jnp.broadcast_to(lax.rsqrt(deg), (RB, N_FEAT))
        dinv_ref[...] = dinv
        hp_ref[...] = dinv * jnp.dot(x_ref[...], w_ref[...],
                                     preferred_element_type=jnp.float32)

    return pl.pallas_call(
        body,
        grid=(NB,),
        in_specs=[_row((RB, NW)), _row((RB, N_FEAT)),
                  _rep((N_FEAT, N_FEAT))],
        out_specs=(_row((RB, N_FEAT)), _row((RB, N_FEAT))),
        out_shape=(
            jax.ShapeDtypeStruct((NP, N_FEAT), jnp.float32),
            jax.ShapeDtypeStruct((NP, N_FEAT), jnp.float32),
        ),
    )(degm, x_p, W1)


def _tc_mid(parts, dinv, b1, W2):
    def body(p_ref, d_ref, b_ref, w_ref, o_ref):
        db = d_ref[...]
        z = db * (p_ref[0] + p_ref[1]) + b_ref[...]
        sp = jnp.maximum(z, 0.0) + jnp.log1p(jnp.exp(-jnp.abs(z)))
        o_ref[...] = db * jnp.dot(sp, w_ref[...],
                                  preferred_element_type=jnp.float32)

    return pl.pallas_call(
        body,
        grid=(NB,),
        in_specs=[_row((NC, RB, N_FEAT)), _row((RB, N_FEAT)),
                  _rep((1, N_FEAT)), _rep((N_FEAT, N_FEAT))],
        out_specs=_row((RB, N_FEAT)),
        out_shape=jax.ShapeDtypeStruct((NP, N_FEAT), jnp.float32),
    )(parts, dinv, b1, W2)


def _tc_final(parts, dinv, b2):
    RF = N_NODES // 10

    def body(p_ref, d_ref, b_ref, o_ref):
        o_ref[...] = d_ref[...] * (p_ref[0] + p_ref[1]) + b_ref[...]

    return pl.pallas_call(
        body,
        grid=(10,),
        in_specs=[_row((NC, RF, N_FEAT)), _row((RF, N_FEAT)),
                  _rep((1, N_FEAT))],
        out_specs=_row((RF, N_FEAT)),
        out_shape=jax.ShapeDtypeStruct((N_NODES, N_FEAT), jnp.float32),
    )(parts, dinv, b2)


def kernel(x, edge_index, W1, b1, W2, b2):
    src = edge_index[0]
    dst = edge_index[1]
    pad = EP - N_EDGES
    pad_src = jnp.arange(pad, dtype=jnp.int32) % N_NODES
    pad_dst = N_NODES + jnp.arange(pad, dtype=jnp.int32) % (NP - N_NODES)
    src_t = jnp.concatenate([src, pad_src]).reshape(NW, CPT, CH)
    dst_t = jnp.concatenate([dst, pad_dst]).reshape(NW, CPT, CH)

    x_p = jnp.pad(x, ((0, NP - N_NODES), (0, 0)))
    b1r = b1.reshape(1, N_FEAT)
    b2r = b2.reshape(1, N_FEAT)

    deg_parts = _sc_degree(dst_t)
    degm = deg_parts.reshape(NW, NP).T
    dinv, h1p = _tc_l1(degm, x_p, W1)
    parts1 = _sc_scatter(h1p, src_t, dst_t)
    h2p = _tc_mid(parts1, dinv, b1r, W2)
    parts2 = _sc_scatter(h2p, src_t, dst_t)
    return _tc_final(parts2, dinv, b2r)
